# baseline (speedup 1.0000x reference)
import functools

import jax
import jax.numpy as jnp
from jax import lax
from jax.experimental import pallas as pl
from jax.experimental.pallas import tpu as pltpu
from jax.experimental.pallas import tpu_sc as plsc

ZMAXPAD = 88
DIM = 64
N_ATOMS = 1048576

NC = 2
NS = 16
NW = NC * NS
B_PER_W = N_ATOMS // NW
CHUNK = 128
N_CHUNKS = B_PER_W // CHUNK
NBUF = 4


def kernel(species, table):
    mesh = plsc.VectorSubcoreMesh(core_axis_name="c", subcore_axis_name="s")

    @functools.partial(
        pl.kernel,
        mesh=mesh,
        compiler_params=pltpu.CompilerParams(use_tc_tiling_on_sc=False),
        out_type=jax.ShapeDtypeStruct((N_ATOMS, DIM), jnp.float32),
        scratch_types=[
            pltpu.VMEM((B_PER_W,), jnp.int32),
            [pltpu.VMEM((CHUNK, DIM), jnp.float32) for _ in range(NBUF)],
            [pltpu.SemaphoreType.DMA for _ in range(NBUF)],
            [pltpu.SemaphoreType.DMA for _ in range(NBUF)],
        ],
    )
    def sc_gather(species_hbm, table_hbm, out_hbm, idx_v, rows, gsems, wsems):
        wid = lax.axis_index("s") * NC + lax.axis_index("c")
        base_row = wid * B_PER_W
        pltpu.sync_copy(species_hbm.at[pl.ds(base_row, B_PER_W)], idx_v)
        my_table = table_hbm.at[wid]

        def body(jj, _):
            base_j = NBUF * jj
            for k in range(NBUF):
                j = base_j + k

                @pl.when(jj > 0)
                def _drain_write():
                    pltpu.make_async_copy(
                        rows[k],
                        out_hbm.at[pl.ds(base_row + j * CHUNK, CHUNK), :],
                        wsems[k]).wait()

                pltpu.async_copy(
                    my_table.at[idx_v.at[pl.ds(j * CHUNK, CHUNK)]],
                    rows[k], gsems[k])

            for k in range(NBUF):
                j = base_j + k
                pltpu.make_async_copy(
                    my_table.at[idx_v.at[pl.ds(j * CHUNK, CHUNK)]],
                    rows[k], gsems[k]).wait()
                pltpu.async_copy(
                    rows[k],
                    out_hbm.at[pl.ds(base_row + j * CHUNK, CHUNK), :],
                    wsems[k])
            return None

        lax.fori_loop(0, N_CHUNKS // NBUF, body, None)
        for k in range(NBUF):
            j = N_CHUNKS - NBUF + k
            pltpu.make_async_copy(
                rows[k],
                out_hbm.at[pl.ds(base_row + j * CHUNK, CHUNK), :],
                wsems[k]).wait()

    table_rep = jnp.tile(table[None], (NW, 1, 1))
    return sc_gather(species, table_rep)
